# baseline (speedup 1.0000x reference)
import functools

import jax
import jax.numpy as jnp
from jax import lax
from jax.experimental import pallas as pl
from jax.experimental.pallas import tpu as pltpu
from jax.experimental.pallas import tpu_sc as plsc

N = 16384
W = 256
LANES = 16
NC = 2
NS = 16
NW = NC * NS
RPW = N // NW
CHUNK = 128
NCHUNK = RPW // CHUNK
WVEC = W // LANES


_GATHER_DNUMS = lax.GatherDimensionNumbers(
    offset_dims=(), collapsed_slice_dims=(0,), start_index_map=(0,))


def _lane_shuffle(x, idx):
    return lax.gather(x, idx[:, None], _GATHER_DNUMS, (1,),
                      mode=lax.GatherScatterMode.PROMISE_IN_BOUNDS)


def _lane_sum(x, lane_iota):
    for sh in (8, 4, 2, 1):
        x = x + _lane_shuffle(x, lane_iota ^ sh)
    return x


def _sc_phase1(mem_hbm, key_hbm, dot_hbm, sq_hbm,
               key_v, buf0, buf1, dot_v, sq_v, sem0, sem1):
    wid = lax.axis_index("s") * NC + lax.axis_index("c")
    base = wid * RPW

    pltpu.sync_copy(key_hbm, key_v)
    kv = [key_v[pl.ds(LANES * j, LANES)] for j in range(WVEC)]

    bufs = (buf0, buf1)
    sems = (sem0, sem1)
    copies = [None, None]
    copies[0] = pltpu.async_copy(mem_hbm.at[pl.ds(base, CHUNK)], buf0, sem0)

    for c in range(NCHUNK):
        cur = c % 2
        if c + 1 < NCHUNK:
            copies[1 - cur] = pltpu.async_copy(
                mem_hbm.at[pl.ds(base + (c + 1) * CHUNK, CHUNK)],
                bufs[1 - cur], sems[1 - cur])
        copies[cur].wait()
        buf = bufs[cur]

        def group_body(g, _, buf=buf, off=c * CHUNK):
            lane_iota = lax.iota(jnp.int32, LANES)
            dvec = jnp.zeros((LANES,), jnp.float32)
            svec = jnp.zeros((LANES,), jnp.float32)
            for i in range(LANES):
                r = g * LANES + i
                v = buf[r, pl.ds(0, LANES)]
                dacc = v * kv[0]
                sacc = v * v
                for j in range(1, WVEC):
                    v = buf[r, pl.ds(LANES * j, LANES)]
                    dacc = dacc + v * kv[j]
                    sacc = sacc + v * v
                dvec = jnp.where(lane_iota == i, _lane_sum(dacc, lane_iota), dvec)
                svec = jnp.where(lane_iota == i, _lane_sum(sacc, lane_iota), svec)
            dot_v[pl.ds(off + g * LANES, LANES)] = dvec
            sq_v[pl.ds(off + g * LANES, LANES)] = svec
            return 0

        lax.fori_loop(0, CHUNK // LANES, group_body, 0)

    pltpu.sync_copy(dot_v, dot_hbm.at[pl.ds(base, RPW)])
    pltpu.sync_copy(sq_v, sq_hbm.at[pl.ds(base, RPW)])


_phase1 = functools.partial(
    pl.kernel,
    out_type=(jax.ShapeDtypeStruct((N,), jnp.float32),
              jax.ShapeDtypeStruct((N,), jnp.float32)),
    mesh=plsc.VectorSubcoreMesh(core_axis_name="c", subcore_axis_name="s"),
    scratch_types=(
        pltpu.VMEM((W,), jnp.float32),
        pltpu.VMEM((CHUNK, W), jnp.float32),
        pltpu.VMEM((CHUNK, W), jnp.float32),
        pltpu.VMEM((RPW,), jnp.float32),
        pltpu.VMEM((RPW,), jnp.float32),
        pltpu.SemaphoreType.DMA,
        pltpu.SemaphoreType.DMA,
    ),
)(_sc_phase1)


def _tc_phase2(dot_ref, sq_ref, key_ref, strength_ref, agate_ref, wgate_ref,
               alloc_ref, out_ref):
    key = key_ref[...]
    key_norm = jnp.sqrt(jnp.sum(key * key))
    dots = dot_ref[...]
    mem_norm = jnp.sqrt(sq_ref[...])
    denom = jnp.maximum(mem_norm * key_norm, 1e-8)
    s = dots / denom * strength_ref[0, 0]
    m = jnp.max(s)
    e = jnp.exp(s - m)
    cw = e / jnp.sum(e)
    ag = agate_ref[0, 0]
    wg = wgate_ref[0, 0]
    out_ref[...] = wg * (ag * alloc_ref[...] + (1.0 - ag) * cw)


def kernel(memory, write_key, write_strength, allocation_gate, write_gate,
           allocation_weighting):
    dots, sumsq = _phase1(memory, write_key)
    out2d = pl.pallas_call(
        _tc_phase2,
        out_shape=jax.ShapeDtypeStruct((128, 128), jnp.float32),
    )(
        dots.reshape(128, 128),
        sumsq.reshape(128, 128),
        write_key.reshape(2, 128),
        write_strength.reshape(1, 1),
        allocation_gate.reshape(1, 1),
        write_gate.reshape(1, 1),
        allocation_weighting.reshape(128, 128),
    )
    return out2d.reshape(N)


# speedup vs baseline: 1.0376x; 1.0376x over previous
import functools

import jax
import jax.numpy as jnp
from jax import lax
from jax.experimental import pallas as pl
from jax.experimental.pallas import tpu as pltpu
from jax.experimental.pallas import tpu_sc as plsc

N = 16384
W = 256
LANES = 16
NC = 2
NS = 16
NW = NC * NS
RPW = N // NW
CHUNK = 128
NCHUNK = RPW // CHUNK
WVEC = W // LANES


_GATHER_DNUMS = lax.GatherDimensionNumbers(
    offset_dims=(), collapsed_slice_dims=(0,), start_index_map=(0,))


def _lane_shuffle(x, idx):
    return lax.gather(x, idx[:, None], _GATHER_DNUMS, (1,),
                      mode=lax.GatherScatterMode.PROMISE_IN_BOUNDS)


def _combine(a, b, sh, lane_iota):
    fa = a + _lane_shuffle(a, lane_iota ^ sh)
    fb = b + _lane_shuffle(b, lane_iota ^ sh)
    return jnp.where((lane_iota & sh) == 0, fa, fb)


def _reduce_batch8(vs, lane_iota):
    for sh in (8, 4, 2):
        vs = [_combine(vs[2 * k], vs[2 * k + 1], sh, lane_iota)
              for k in range(len(vs) // 2)]
    return vs[0]


def _bitrev4(lane_iota):
    return (((lane_iota & 1) << 3) | ((lane_iota & 2) << 1)
            | ((lane_iota & 4) >> 1) | ((lane_iota & 8) >> 3))


def _finish16(za, zb, lane_iota, bitrev):
    z = _combine(za, zb, 1, lane_iota)
    return _lane_shuffle(z, bitrev)


def _sc_phase1(mem_hbm, key_hbm, dot_hbm, sq_hbm,
               key_v, buf0, buf1, dot_v, sq_v, sem0, sem1):
    wid = lax.axis_index("s") * NC + lax.axis_index("c")
    base = wid * RPW

    pltpu.sync_copy(key_hbm, key_v)
    kv = [key_v[pl.ds(LANES * j, LANES)] for j in range(WVEC)]

    bufs = (buf0, buf1)
    sems = (sem0, sem1)
    copies = [None, None]
    copies[0] = pltpu.async_copy(mem_hbm.at[pl.ds(base, CHUNK)], buf0, sem0)

    for c in range(NCHUNK):
        cur = c % 2
        if c + 1 < NCHUNK:
            copies[1 - cur] = pltpu.async_copy(
                mem_hbm.at[pl.ds(base + (c + 1) * CHUNK, CHUNK)],
                bufs[1 - cur], sems[1 - cur])
        copies[cur].wait()
        buf = bufs[cur]

        def group_body(g, _, buf=buf, off=c * CHUNK):
            lane_iota = lax.iota(jnp.int32, LANES)
            bitrev = _bitrev4(lane_iota)
            zd, zs = [], []
            for batch in range(2):
                daccs, saccs = [], []
                for i in range(8):
                    r = g * LANES + batch * 8 + i
                    v = buf[r, pl.ds(0, LANES)]
                    dacc = v * kv[0]
                    sacc = v * v
                    for j in range(1, WVEC):
                        v = buf[r, pl.ds(LANES * j, LANES)]
                        dacc = dacc + v * kv[j]
                        sacc = sacc + v * v
                    daccs.append(dacc)
                    saccs.append(sacc)
                zd.append(_reduce_batch8(daccs, lane_iota))
                zs.append(_reduce_batch8(saccs, lane_iota))
            dot_v[pl.ds(off + g * LANES, LANES)] = _finish16(
                zd[0], zd[1], lane_iota, bitrev)
            sq_v[pl.ds(off + g * LANES, LANES)] = _finish16(
                zs[0], zs[1], lane_iota, bitrev)
            return 0

        lax.fori_loop(0, CHUNK // LANES, group_body, 0)

    pltpu.sync_copy(dot_v, dot_hbm.at[pl.ds(base, RPW)])
    pltpu.sync_copy(sq_v, sq_hbm.at[pl.ds(base, RPW)])


_phase1 = functools.partial(
    pl.kernel,
    out_type=(jax.ShapeDtypeStruct((N,), jnp.float32),
              jax.ShapeDtypeStruct((N,), jnp.float32)),
    mesh=plsc.VectorSubcoreMesh(core_axis_name="c", subcore_axis_name="s"),
    scratch_types=(
        pltpu.VMEM((W,), jnp.float32),
        pltpu.VMEM((CHUNK, W), jnp.float32),
        pltpu.VMEM((CHUNK, W), jnp.float32),
        pltpu.VMEM((RPW,), jnp.float32),
        pltpu.VMEM((RPW,), jnp.float32),
        pltpu.SemaphoreType.DMA,
        pltpu.SemaphoreType.DMA,
    ),
)(_sc_phase1)


def _tc_phase2(dot_ref, sq_ref, key_ref, strength_ref, agate_ref, wgate_ref,
               alloc_ref, out_ref):
    key = key_ref[...]
    key_norm = jnp.sqrt(jnp.sum(key * key))
    dots = dot_ref[...]
    mem_norm = jnp.sqrt(sq_ref[...])
    denom = jnp.maximum(mem_norm * key_norm, 1e-8)
    s = dots / denom * strength_ref[0, 0]
    m = jnp.max(s)
    e = jnp.exp(s - m)
    cw = e / jnp.sum(e)
    ag = agate_ref[0, 0]
    wg = wgate_ref[0, 0]
    out_ref[...] = wg * (ag * alloc_ref[...] + (1.0 - ag) * cw)


def kernel(memory, write_key, write_strength, allocation_gate, write_gate,
           allocation_weighting):
    dots, sumsq = _phase1(memory, write_key)
    out2d = pl.pallas_call(
        _tc_phase2,
        out_shape=jax.ShapeDtypeStruct((128, 128), jnp.float32),
    )(
        dots.reshape(128, 128),
        sumsq.reshape(128, 128),
        write_key.reshape(2, 128),
        write_strength.reshape(1, 1),
        allocation_gate.reshape(1, 1),
        write_gate.reshape(1, 1),
        allocation_weighting.reshape(128, 128),
    )
    return out2d.reshape(N)
